# baseline (speedup 1.0000x reference)
import functools

import jax
import jax.numpy as jnp
from jax import lax
from jax.experimental import pallas as pl
from jax.experimental.pallas import tpu as pltpu
from jax.experimental.pallas import tpu_sc as plsc

B = 16384
D = 32
NC = 2
NS = 16
L = 16
NW = NC * NS
BPW = B // NW
G = BPW // L


def _body(uidx_hbm, iidx_hbm, table_hbm, out_hbm,
          uidx_v, iidx_v, urows_v, irows_v, out_v, sem_u, sem_i):
    wid = lax.axis_index("s") * NC + lax.axis_index("c")
    base = wid * BPW

    pltpu.sync_copy(uidx_hbm.at[pl.ds(base, BPW)], uidx_v)
    pltpu.sync_copy(iidx_hbm.at[pl.ds(base, BPW)], iidx_v)

    cu = pltpu.async_copy(table_hbm.at[uidx_v], urows_v, sem_u)
    ci = pltpu.async_copy(table_hbm.at[iidx_v], irows_v, sem_i)
    cu.wait()
    ci.wait()

    lane = lax.iota(jnp.int32, L)

    def group(g, carry):
        rid = g * L + lane
        acc = jnp.zeros((L,), jnp.float32)
        for d in range(D):
            cid = jnp.full((L,), d, jnp.int32)
            uv = plsc.load_gather(urows_v, [rid, cid])
            iv = plsc.load_gather(irows_v, [rid, cid])
            acc = acc + uv * iv
        out_v[pl.ds(g * L, L)] = acc
        return carry

    lax.fori_loop(0, G, group, 0)

    pltpu.sync_copy(out_v, out_hbm.at[pl.ds(base, BPW)])


def kernel(edge_index, edge_label_index, embedding_weight):
    del edge_index
    uidx = edge_label_index[0]
    iidx = edge_label_index[1]
    mesh = plsc.VectorSubcoreMesh(core_axis_name="c", subcore_axis_name="s")
    f = pl.kernel(
        _body,
        out_type=jax.ShapeDtypeStruct((B,), jnp.float32),
        mesh=mesh,
        compiler_params=pltpu.CompilerParams(
            needs_layout_passes=False, use_tc_tiling_on_sc=False
        ),
        scratch_types=[
            pltpu.VMEM((BPW,), jnp.int32),
            pltpu.VMEM((BPW,), jnp.int32),
            pltpu.VMEM((BPW, D), jnp.float32),
            pltpu.VMEM((BPW, D), jnp.float32),
            pltpu.VMEM((BPW,), jnp.float32),
            pltpu.SemaphoreType.DMA,
            pltpu.SemaphoreType.DMA,
        ],
    )
    return f(uidx, iidx, embedding_weight)
